# baseline (speedup 1.0000x reference)
import dataclasses
import functools

import jax
import jax.numpy as jnp
from jax import lax
from jax.experimental import pallas as pl
from jax.experimental.pallas import tpu as pltpu
from jax.experimental.pallas import tpu_sc as plsc

NUM_VIRTUAL = 1000
EMBED_DIM = 128
_NC = 2
_NS = 16
_NW = _NC * _NS
_LANES = 16
_CHUNK = 128


@functools.lru_cache(maxsize=None)
def _make_sc_lookup(num_flat: int):
    assert num_flat % (_NW * _CHUNK) == 0
    b_per_w = num_flat // _NW
    n_chunks = b_per_w // _CHUNK
    mesh = plsc.VectorSubcoreMesh(core_axis_name="c", subcore_axis_name="s")
    cp = pltpu.CompilerParams()
    if "needs_layout_passes" in pltpu.CompilerParams.__dataclass_fields__:
        cp = dataclasses.replace(cp, needs_layout_passes=False)

    @functools.partial(
        pl.kernel,
        compiler_params=cp,
        out_type=jax.ShapeDtypeStruct((num_flat, EMBED_DIM), jnp.float32),
        mesh=mesh,
        scratch_types=[
            pltpu.VMEM((b_per_w,), jnp.int32),
            pltpu.VMEM((_LANES,), jnp.int32),
            pltpu.VMEM((_CHUNK, EMBED_DIM), jnp.float32),
            pltpu.VMEM((_CHUNK, EMBED_DIM), jnp.float32),
            pltpu.VMEM((b_per_w,), jnp.int32),
            pltpu.VMEM((n_chunks, _CHUNK), jnp.int32),
            pltpu.SemaphoreType.DMA,
        ],
    )
    def sc_lookup(ids_hbm, start_hbm, table_hbm, zrows_hbm, out_hbm,
                  idx_v, start_v, zero_buf, staging, row_cp, pos_cp, zsem):
        wid = lax.axis_index("s") * _NC + lax.axis_index("c")
        base = wid * b_per_w

        pltpu.sync_copy(zrows_hbm, zero_buf)
        zwrites = [
            pltpu.async_copy(
                zero_buf, out_hbm.at[pl.ds(base + g * _CHUNK, _CHUNK)], zsem)
            for g in range(n_chunks)
        ]

        pltpu.sync_copy(ids_hbm.at[pl.ds(base, b_per_w)], idx_v)
        pltpu.sync_copy(start_hbm, start_v)
        sv = start_v[...]
        nv = jnp.full((_LANES,), NUM_VIRTUAL, jnp.int32)
        lane = lax.iota(jnp.int32, _LANES)

        def body(it, carry):
            k, safe = carry
            off = it * _LANES
            v = idx_v[pl.ds(off, _LANES)]
            m = v >= sv
            d = jnp.minimum(v - sv, nv - 1)
            pos = lane + (base + off)
            c = jnp.cumsum(m.astype(jnp.int32))
            t = jnp.maximum(k + c - 1, 0)
            plsc.store_scatter(row_cp, [t], d, mask=m)
            plsc.store_scatter(
                pos_cp, [t >> 7, t & (_CHUNK - 1)], pos, mask=m)
            nvpos = jnp.max(jnp.where(m, -1, pos))
            return k + jnp.max(c), jnp.where(nvpos >= 0, nvpos, safe)

        k, safe = lax.fori_loop(
            0, b_per_w // _LANES, body,
            (jnp.int32(0), base), unroll=False)

        g_base = (k >> 7) << 7
        safe_v = lane * 0 + safe
        for j in range(_CHUNK // _LANES):
            off = g_base + j * _LANES + lane
            sel = jnp.logical_and(off >= k, off < b_per_w)
            offc = jnp.minimum(off, b_per_w - 1)
            plsc.store_scatter(row_cp, [offc], lane * 0, mask=sel)
            plsc.store_scatter(
                pos_cp, [offc >> 7, offc & (_CHUNK - 1)], safe_v, mask=sel)

        for cp in zwrites:
            cp.wait()

        z16 = jnp.zeros((_LANES,), jnp.float32)
        for g in range(n_chunks):
            @pl.when(g * _CHUNK < k)
            def _():
                pltpu.sync_copy(
                    table_hbm.at[row_cp.at[pl.ds(g * _CHUNK, _CHUNK)]],
                    staging)
                k_local = k - g * _CHUNK

                @pl.when(k_local < _CHUNK)
                def _():
                    @pl.loop(0, _CHUNK)
                    def _(r):
                        @pl.when(r >= k_local)
                        def _():
                            row_ref = staging.at[r]
                            for col in range(EMBED_DIM // _LANES):
                                row_ref[pl.ds(col * _LANES, _LANES)] = z16

                pltpu.sync_copy(staging, out_hbm.at[pos_cp.at[g]])

    return sc_lookup


def kernel(token_ids, virtual_token_start_idx, virtual_embeddings):
    original_shape = token_ids.shape
    flat = token_ids.reshape(-1)
    if flat.dtype != jnp.int32:
        flat = flat.astype(jnp.int32)
    num_flat = flat.shape[0]
    table = virtual_embeddings.astype(jnp.float32)
    start = jnp.full((_LANES,), virtual_token_start_idx, jnp.int32)
    zrows = jnp.zeros((_CHUNK, EMBED_DIM), jnp.float32)
    out = _make_sc_lookup(num_flat)(flat, start, table, zrows)
    return out.reshape(*original_shape, EMBED_DIM)


# speedup vs baseline: 2.0747x; 2.0747x over previous
import dataclasses
import functools

import jax
import jax.numpy as jnp
from jax import lax
from jax.experimental import pallas as pl
from jax.experimental.pallas import tpu as pltpu
from jax.experimental.pallas import tpu_sc as plsc

NUM_VIRTUAL = 1000
EMBED_DIM = 128
_NC = 2
_NS = 16
_NW = _NC * _NS
_LANES = 16
_CHUNK = 128


@functools.lru_cache(maxsize=None)
def _make_sc_lookup(num_flat: int):
    assert num_flat % (_NW * _CHUNK) == 0
    b_per_w = num_flat // _NW
    n_chunks = b_per_w // _CHUNK
    mesh = plsc.VectorSubcoreMesh(core_axis_name="c", subcore_axis_name="s")
    cp = pltpu.CompilerParams()
    if "needs_layout_passes" in pltpu.CompilerParams.__dataclass_fields__:
        cp = dataclasses.replace(cp, needs_layout_passes=False)

    @functools.partial(
        pl.kernel,
        compiler_params=cp,
        out_type=jax.ShapeDtypeStruct((num_flat, EMBED_DIM), jnp.float32),
        mesh=mesh,
        scratch_types=[
            pltpu.VMEM((b_per_w,), jnp.int32),
            pltpu.VMEM((_LANES,), jnp.int32),
            pltpu.VMEM((_CHUNK, EMBED_DIM), jnp.float32),
            pltpu.VMEM((_CHUNK, EMBED_DIM), jnp.float32),
            pltpu.VMEM((b_per_w,), jnp.int32),
            pltpu.VMEM((n_chunks, _CHUNK), jnp.int32),
            pltpu.SemaphoreType.DMA,
        ],
    )
    def sc_lookup(ids_hbm, start_hbm, table_hbm, zrows_hbm, out_hbm,
                  idx_v, start_v, zero_buf, staging, row_cp, pos_cp, zsem):
        wid = lax.axis_index("s") * _NC + lax.axis_index("c")
        base = wid * b_per_w

        pltpu.sync_copy(zrows_hbm, zero_buf)
        zwrites = [
            pltpu.async_copy(
                zero_buf, out_hbm.at[pl.ds(base + g * _CHUNK, _CHUNK)], zsem)
            for g in range(n_chunks)
        ]

        pltpu.sync_copy(ids_hbm.at[pl.ds(base, b_per_w)], idx_v)
        pltpu.sync_copy(start_hbm, start_v)
        sv = start_v[...]
        nv = jnp.full((_LANES,), NUM_VIRTUAL, jnp.int32)
        lane = lax.iota(jnp.int32, _LANES)

        def body(it, carry):
            k, safe = carry
            off = it * _LANES
            v = idx_v[pl.ds(off, _LANES)]
            m = v >= sv
            d = jnp.minimum(v - sv, nv - 1)
            pos = lane + (base + off)
            c = jnp.cumsum(m.astype(jnp.int32))
            t = jnp.maximum(k + c - 1, 0)
            plsc.store_scatter(row_cp, [t], d, mask=m)
            plsc.store_scatter(
                pos_cp, [t >> 7, t & (_CHUNK - 1)], pos, mask=m)
            nvpos = jnp.max(jnp.where(m, -1, pos))
            return k + jnp.max(c), jnp.where(nvpos >= 0, nvpos, safe)

        k, safe = lax.fori_loop(
            0, b_per_w // _LANES, body,
            (jnp.int32(0), base), unroll=False)

        g_base = (k >> 7) << 7
        safe_v = lane * 0 + safe
        for j in range(_CHUNK // _LANES):
            off = g_base + j * _LANES + lane
            sel = jnp.logical_and(off >= k, off < b_per_w)
            offc = jnp.minimum(off, b_per_w - 1)
            plsc.store_scatter(row_cp, [offc], off & 511, mask=sel)
            plsc.store_scatter(
                pos_cp, [offc >> 7, offc & (_CHUNK - 1)], safe_v, mask=sel)

        for cp in zwrites:
            cp.wait()

        z16 = jnp.zeros((_LANES,), jnp.float32)
        for g in range(n_chunks):
            @pl.when(g * _CHUNK < k)
            def _():
                pltpu.sync_copy(
                    table_hbm.at[row_cp.at[pl.ds(g * _CHUNK, _CHUNK)]],
                    staging)
                k_local = k - g * _CHUNK

                @pl.when(k_local < _CHUNK)
                def _():
                    @pl.loop(0, _CHUNK)
                    def _(r):
                        @pl.when(r >= k_local)
                        def _():
                            row_ref = staging.at[r]
                            for col in range(EMBED_DIM // _LANES):
                                row_ref[pl.ds(col * _LANES, _LANES)] = z16

                pltpu.sync_copy(staging, out_hbm.at[pos_cp.at[g]])

    return sc_lookup


def kernel(token_ids, virtual_token_start_idx, virtual_embeddings):
    original_shape = token_ids.shape
    flat = token_ids.reshape(-1)
    if flat.dtype != jnp.int32:
        flat = flat.astype(jnp.int32)
    num_flat = flat.shape[0]
    table = virtual_embeddings.astype(jnp.float32)
    start = jnp.full((_LANES,), virtual_token_start_idx, jnp.int32)
    zrows = jnp.zeros((_CHUNK, EMBED_DIM), jnp.float32)
    out = _make_sc_lookup(num_flat)(flat, start, table, zrows)
    return out.reshape(*original_shape, EMBED_DIM)


# speedup vs baseline: 2.1151x; 1.0195x over previous
import dataclasses
import functools

import jax
import jax.numpy as jnp
from jax import lax
from jax.experimental import pallas as pl
from jax.experimental.pallas import tpu as pltpu
from jax.experimental.pallas import tpu_sc as plsc

NUM_VIRTUAL = 1000
EMBED_DIM = 128
_NC = 2
_NS = 16
_NW = _NC * _NS
_LANES = 16
_CHUNK = 128


@functools.lru_cache(maxsize=None)
def _make_sc_lookup(num_flat: int):
    assert num_flat % (_NW * _CHUNK) == 0
    b_per_w = num_flat // _NW
    n_chunks = b_per_w // _CHUNK
    mesh = plsc.VectorSubcoreMesh(core_axis_name="c", subcore_axis_name="s")
    cp = pltpu.CompilerParams()
    if "needs_layout_passes" in pltpu.CompilerParams.__dataclass_fields__:
        cp = dataclasses.replace(cp, needs_layout_passes=False)

    @functools.partial(
        pl.kernel,
        compiler_params=cp,
        out_type=jax.ShapeDtypeStruct((num_flat, EMBED_DIM), jnp.float32),
        mesh=mesh,
        scratch_types=[
            pltpu.VMEM((b_per_w,), jnp.int32),
            pltpu.VMEM((_LANES,), jnp.int32),
            pltpu.VMEM((_CHUNK, EMBED_DIM), jnp.float32),
            pltpu.VMEM((_CHUNK, EMBED_DIM), jnp.float32),
            pltpu.VMEM((b_per_w,), jnp.int32),
            pltpu.VMEM((n_chunks, _CHUNK), jnp.int32),
            pltpu.SemaphoreType.DMA,
        ],
    )
    def sc_lookup(ids_hbm, start_hbm, table_hbm, zrows_hbm, out_hbm,
                  idx_v, start_v, zero_buf, staging, row_cp, pos_cp, zsem):
        wid = lax.axis_index("s") * _NC + lax.axis_index("c")
        base = wid * b_per_w

        pltpu.sync_copy(zrows_hbm, zero_buf)

        @pl.loop(0, n_chunks)
        def _(g):
            pltpu.async_copy(
                zero_buf, out_hbm.at[pl.ds(base + g * _CHUNK, _CHUNK)], zsem)

        pltpu.sync_copy(ids_hbm.at[pl.ds(base, b_per_w)], idx_v)
        pltpu.sync_copy(start_hbm, start_v)
        sv = start_v[...]
        nv = jnp.full((_LANES,), NUM_VIRTUAL, jnp.int32)
        lane = lax.iota(jnp.int32, _LANES)

        def body(it, carry):
            k, safe = carry
            off = it * _LANES
            v = idx_v[pl.ds(off, _LANES)]
            m = v >= sv
            d = jnp.minimum(v - sv, nv - 1)
            pos = lane + (base + off)
            c = jnp.cumsum(m.astype(jnp.int32))
            t = jnp.maximum(k + c - 1, 0)
            plsc.store_scatter(row_cp, [t], d, mask=m)
            plsc.store_scatter(
                pos_cp, [t >> 7, t & (_CHUNK - 1)], pos, mask=m)
            nvpos = jnp.max(jnp.where(m, -1, pos))
            return k + jnp.max(c), jnp.where(nvpos >= 0, nvpos, safe)

        k, safe = lax.fori_loop(
            0, b_per_w // _LANES, body,
            (jnp.int32(0), base), unroll=False)

        g_base = (k >> 7) << 7
        safe_v = lane * 0 + safe
        for j in range(_CHUNK // _LANES):
            off = g_base + j * _LANES + lane
            sel = jnp.logical_and(off >= k, off < b_per_w)
            offc = jnp.minimum(off, b_per_w - 1)
            plsc.store_scatter(row_cp, [offc], off & 511, mask=sel)
            plsc.store_scatter(
                pos_cp, [offc >> 7, offc & (_CHUNK - 1)], safe_v, mask=sel)

        z16 = jnp.zeros((_LANES,), jnp.float32)

        def gather_group(g):
            pltpu.sync_copy(
                table_hbm.at[row_cp.at[pl.ds(g * _CHUNK, _CHUNK)]], staging)
            k_local = k - g * _CHUNK

            @pl.when(k_local < _CHUNK)
            def _():
                @pl.loop(0, _CHUNK)
                def _(r):
                    @pl.when(r >= k_local)
                    def _():
                        row_ref = staging.at[r]
                        for col in range(EMBED_DIM // _LANES):
                            row_ref[pl.ds(col * _LANES, _LANES)] = z16

        @pl.when(k > 0)
        def _():
            gather_group(0)

        @pl.loop(0, n_chunks)
        def _(g):
            pltpu.make_async_copy(
                zero_buf, out_hbm.at[pl.ds(base + g * _CHUNK, _CHUNK)],
                zsem).wait()

        @pl.when(k > 0)
        def _():
            pltpu.sync_copy(staging, out_hbm.at[pos_cp.at[0]])

        for g in range(1, n_chunks):
            @pl.when(g * _CHUNK < k)
            def _():
                gather_group(g)
                pltpu.sync_copy(staging, out_hbm.at[pos_cp.at[g]])

    return sc_lookup


def kernel(token_ids, virtual_token_start_idx, virtual_embeddings):
    original_shape = token_ids.shape
    flat = token_ids.reshape(-1)
    if flat.dtype != jnp.int32:
        flat = flat.astype(jnp.int32)
    num_flat = flat.shape[0]
    table = virtual_embeddings.astype(jnp.float32)
    start = jnp.full((_LANES,), virtual_token_start_idx, jnp.int32)
    zrows = jnp.zeros((_CHUNK, EMBED_DIM), jnp.float32)
    out = _make_sc_lookup(num_flat)(flat, start, table, zrows)
    return out.reshape(*original_shape, EMBED_DIM)
